# baseline (speedup 1.0000x reference)
import jax
import jax.numpy as jnp
from jax import lax
from jax.experimental import pallas as pl

_BS, _NR, _NX = 8, 64, 32
_NN = _NR * _NX
_CR = 8
_CN = _CR * _NX
_GU = _NR // _CR
_BB = 1
_NEG = -1e30

_F32 = jnp.float32


def _mm(a, b):
    return jnp.dot(a, b, preferred_element_type=_F32)


def _mm_t(w, x):
    return lax.dot_general(w, x, (((0,), (1,)), ((), ())),
                           preferred_element_type=_F32)


def _mm_rt(a, w):
    return lax.dot_general(a, w, (((1,), (1,)), ((), ())),
                           preferred_element_type=_F32)


def _d2v_body(pairs_ref,
              f1w, f1b, f2w, f2b, f3w, f3b, f4w, f4b, f5w, f5b,
              g1w, g1b, g2w, g2b,
              h1w, h1b, h2w, h2b, h3w, h3b, h4w, h4b, h5w, h5b,
              wa1w, wa1b, wa2w, wa2b, wb1w, wb1b, wb2w, wb2b,
              w0_out, w1_out):
    relu = lambda v: jnp.maximum(v, 0.0)
    x = pairs_ref[...]
    x = relu(_mm_rt(x, f1w[...]) + f1b[...])
    x = x + relu(_mm_rt(x, f2w[...]) + f2b[...])
    x = x + relu(_mm_rt(x, f3w[...]) + f3b[...])
    x = x + relu(_mm_rt(x, f4w[...]) + f4b[...])
    x = relu(_mm_rt(x, f5w[...]) + f5b[...])
    x = jnp.mean(x.reshape(_BS * _NR, _NX, 64), axis=1)
    x = relu(_mm_rt(x, g1w[...]) + g1b[...])
    x = relu(_mm_rt(x, g2w[...]) + g2b[...])
    x = jnp.mean(x.reshape(_BS, _NR, 64), axis=1)
    x = relu(_mm_rt(x, h1w[...]) + h1b[...])
    x = x + relu(_mm_rt(x, h2w[...]) + h2b[...])
    x = x + relu(_mm_rt(x, h3w[...]) + h3b[...])
    x = x + relu(_mm_rt(x, h4w[...]) + h4b[...])
    d2v = relu(_mm_rt(x, h5w[...]) + h5b[...])
    w0_out[...] = _mm_rt(relu(_mm_rt(d2v, wa1w[...]) + wa1b[...]),
                         wa2w[...]) + wa2b[...]
    w1_out[...] = _mm_rt(relu(_mm_rt(d2v, wb1w[...]) + wb1b[...]),
                         wb2w[...]) + wb2b[...]


def _softmax_rows(s):
    m = jnp.max(s, axis=-1, keepdims=True)
    e = jnp.exp(s - m)
    return e / (jnp.sum(e, axis=-1, keepdims=True) + 1e-16)


def _attend(h, a_st, a_d, out, masked):
    s_parts = []
    for i in range(len(h)):
        for u in range(_GU):
            for g in range(2):
                row = a_st[i][g][0:1, u * _CN:(u + 1) * _CN]
                col = a_d[i][g][u * _CN:(u + 1) * _CN, 0:1]
                s_parts.append(row + col)
    coef = _softmax_rows(masked(jnp.concatenate(s_parts, axis=0)))
    outs = []
    for i in range(len(h)):
        x_parts = []
        for u in range(_GU):
            aggs = [
                _mm(coef[((i * _GU + u) * 2 + g) * _CN:
                         ((i * _GU + u) * 2 + g + 1) * _CN, :],
                    h[i][u * _CN:(u + 1) * _CN, g * out:(g + 1) * out])
                for g in range(2)]
            x_parts.append(jnp.concatenate(aggs, axis=-1))
        outs.append(jnp.concatenate(x_parts, axis=0))
    return outs


def _gnn_body(xcol_ref, w0c0_ref, w0_ref, w1_ref, lw1_ref,
              wo_ref, bo_ref, out_ref):
    nw = _BB * _GU * 2 * _CN
    ii = (lax.broadcasted_iota(jnp.int32, (nw, _CN), 0) % _CN) // _NX
    jj = lax.broadcasted_iota(jnp.int32, (nw, _CN), 1) // _NX
    mask = ii == jj

    def masked(s):
        s = jnp.where(s >= 0.0, s, 0.2 * s)
        return jnp.where(mask, s, _NEG)

    w0 = [w0_ref[i] for i in range(_BB)]
    w1 = [w1_ref[i] for i in range(_BB)]
    h = [xcol_ref[i] * w0c0_ref[i] for i in range(_BB)]
    a_st, a_d = [], []
    for i in range(_BB):
        si, di = [], []
        for g in range(2):
            hg = h[i][:, 32 * g:32 * (g + 1)]
            si.append(_mm_rt(w0[i][0:1, 128 + 32 * g:160 + 32 * g], hg))
            di.append(_mm_rt(hg, w0[i][0:1, 192 + 32 * g:224 + 32 * g]))
        a_st.append(si)
        a_d.append(di)
    x2 = [xi + w0[i][0:1, 256:320]
          for i, xi in enumerate(_attend(h, a_st, a_d, 32, masked))]

    h2 = [_mm_rt(x2[i], lw1_ref[i]) for i in range(_BB)]
    a_st2, a_d2 = [], []
    for i in range(_BB):
        si, di = [], []
        for g in range(2):
            h2g = h2[i][:, 8 * g:8 * (g + 1)]
            si.append(_mm_rt(w1[i][0:1, 1024 + 8 * g:1032 + 8 * g], h2g))
            di.append(_mm_rt(h2g, w1[i][0:1, 1040 + 8 * g:1048 + 8 * g]))
        a_st2.append(si)
        a_d2.append(di)
    x3 = [xi + w1[i][0:1, 1056:1072]
          for i, xi in enumerate(_attend(h2, a_st2, a_d2, 8, masked))]

    for i in range(_BB):
        row = jnp.sum(x3[i].reshape(_NR, _NX, 16), axis=1)
        out_ref[i] = _mm_rt(row, wo_ref[...]) + bo_ref[...]


def _full(shape):
    nd = len(shape)
    return pl.BlockSpec(shape, lambda b: (0,) * nd)


def _per_batch(shape):
    nd = len(shape)
    return pl.BlockSpec((1,) + shape, lambda b: (b,) + (0,) * nd)


@jax.jit
def kernel(xs, pairs, params):
    p = params

    def wt(name):
        W, b = p[name]
        return W.astype(_F32), b.reshape(1, -1).astype(_F32)

    names = ["f1", "f2r", "f3r", "f4r", "f5", "g1", "g2",
             "h1", "h2r", "h3r", "h4r", "h5",
             "wg0_1", "wg0_2", "wg1_1", "wg1_2"]
    wargs = []
    for n in names:
        W, b = wt(n)
        wargs += [W, b]

    pairs_f = pairs.reshape(_BS * _NN, 2)
    w0, w1 = pl.pallas_call(
        _d2v_body,
        out_shape=[jax.ShapeDtypeStruct((_BS, 320), _F32),
                   jax.ShapeDtypeStruct((_BS, 1072), _F32)],
    )(pairs_f, *wargs)

    w0c0 = w0[:, 0:128:2].reshape(_BS, 1, 64)
    lin_w1 = w1[:, :1024].reshape(_BS, 16, 64)

    Wo, bo = p["out_lin"]
    Wo = Wo.astype(_F32)
    bo = bo.reshape(1, 2).astype(_F32)

    xcol = xs.reshape(_BS, _NN, 1).astype(_F32)
    w0r = w0.reshape(_BS, 1, 320)
    w1r = w1.reshape(_BS, 1, 1072)

    def _per_bb(shape):
        nd = len(shape)
        return pl.BlockSpec((_BB,) + shape, lambda b: (b,) + (0,) * nd)

    out = pl.pallas_call(
        _gnn_body,
        grid=(_BS // _BB,),
        in_specs=[_per_bb((_NN, 1)), _per_bb((1, 64)),
                  _per_bb((1, 320)), _per_bb((1, 1072)),
                  _per_bb((16, 64)),
                  _full((2, 16)), _full((1, 2))],
        out_specs=_per_bb((_NR, 2)),
        out_shape=jax.ShapeDtypeStruct((_BS, _NR, 2), _F32),
    )(xcol, w0c0, w0r, w1r, lin_w1, Wo, bo)
    return out
